# Initial kernel scaffold; baseline (speedup 1.0000x reference)
import jax
import jax.numpy as jnp
from jax.experimental import pallas as pl


def kernel(user_inputs, item_inputs, support_rows, support_cols, support_vals, weight):
    raise NotImplementedError("write your pallas kernel here")


# speedup vs baseline: 8.5531x; 8.5531x over previous
import functools

import jax
import jax.numpy as jnp
from jax import lax
from jax.experimental import pallas as pl
from jax.experimental.pallas import tpu as pltpu
from jax.experimental.pallas import tpu_sc as plsc

_N = 10000
_DIN = 128
_DOUT = 160
_NS = 5
_DC = _DOUT // _NS
_E = 100000

_NSUB = 16
_CHUNK = 1024
_KIDX = _CHUNK // 128
_EP = 102400
_NBLK = _EP // _CHUNK
_BPT = -(-_NBLK // _NSUB)
_PASS_LVLS = (3, 2)
_ACC_ROWS = max(_PASS_LVLS) * _N


def _mm_body(x_ref, w_ref, o_ref):
    o_ref[...] = jnp.dot(
        x_ref[0], w_ref[...], preferred_element_type=jnp.float32
    )[None]


def _tables(x, w):
    return pl.pallas_call(
        _mm_body,
        grid=(2,),
        in_specs=[
            pl.BlockSpec((1, _N, _DIN), lambda d: (d, 0, 0)),
            pl.BlockSpec((_DIN, _DOUT), lambda d: (0, 0)),
        ],
        out_specs=pl.BlockSpec((1, _N, _DOUT), lambda d: (d, 0, 0)),
        out_shape=jax.ShapeDtypeStruct((2, _N, _DOUT), jnp.float32),
    )(x, w)


@functools.partial(
    pl.kernel,
    out_type=jax.ShapeDtypeStruct((2, _NS * _N, _DC), jnp.float32),
    mesh=plsc.VectorSubcoreMesh(core_axis_name="c", subcore_axis_name="s"),
    compiler_params=pltpu.CompilerParams(use_tc_tiling_on_sc=False),
    scratch_types=[
        pltpu.VMEM((_KIDX, 128), jnp.int32),
        pltpu.VMEM((_KIDX, 128), jnp.int32),
        pltpu.VMEM((_CHUNK,), jnp.float32),
        pltpu.VMEM((_CHUNK, _DC), jnp.float32),
        pltpu.VMEM_SHARED((_ACC_ROWS, _DC), jnp.float32),
        pltpu.SemaphoreType.DMA,
    ],
)
def _sc_aggregate(table, gidx, sidx, vals, zeros, out,
                  gi_v, si_v, vv, rows_v, acc, sem):
    d = lax.axis_index("c")
    s = lax.axis_index("s")

    base_lvl = 0
    out_row0 = 0
    for nlvl in _PASS_LVLS:
        nrows = nlvl * _N
        stripe = nrows // _NSUB

        pltpu.sync_copy(zeros.at[pl.ds(0, stripe)],
                        acc.at[pl.ds(s * stripe, stripe)])
        plsc.subcore_barrier()

        def chunk_body(t, carry, base_lvl=base_lvl):
            i = base_lvl + t // _BPT
            q = t - (t // _BPT) * _BPT
            b = s + q * _NSUB

            @pl.when(b < _NBLK)
            def _():
                base = b * _CHUNK
                krow = b * _KIDX
                pltpu.sync_copy(gidx.at[d, i, pl.ds(krow, _KIDX)], gi_v)
                pltpu.sync_copy(sidx.at[d, i, pl.ds(krow, _KIDX)], si_v)
                pltpu.sync_copy(vals.at[pl.ds(i * _EP + base, _CHUNK)], vv)

                cps = [
                    pltpu.async_copy(
                        table.at[gi_v.at[j]],
                        rows_v.at[pl.ds(j * 128, 128)],
                        sem,
                    )
                    for j in range(_KIDX)
                ]
                for cp in cps:
                    cp.wait()

                def g_body(g, c):
                    vv16 = vv[pl.ds(g * 16, 16)]
                    e0 = g * 16
                    for k in range(16):
                        v = vv16[k]
                        rows_v[e0 + k, pl.ds(0, 16)] = (
                            rows_v[e0 + k, pl.ds(0, 16)] * v)
                        rows_v[e0 + k, pl.ds(16, 16)] = (
                            rows_v[e0 + k, pl.ds(16, 16)] * v)
                    return c

                lax.fori_loop(0, _CHUNK // 16, g_body, 0)

                for j in range(_KIDX):
                    pltpu.sync_copy(
                        rows_v.at[pl.ds(j * 128, 128)],
                        acc.at[si_v.at[j]],
                        add=True,
                    )
            return carry

        lax.fori_loop(0, nlvl * _BPT, chunk_body, 0)

        plsc.subcore_barrier()
        pltpu.sync_copy(
            acc.at[pl.ds(s * stripe, stripe)],
            out.at[d, pl.ds(out_row0 + s * stripe, stripe)],
        )
        plsc.subcore_barrier()

        base_lvl += nlvl
        out_row0 += nrows


def kernel(user_inputs, item_inputs, support_rows, support_cols,
           support_vals, weight):
    x = jnp.stack([item_inputs, user_inputs])
    table = _tables(x, weight).reshape(2 * _N * _NS, _DC)

    pad = _EP - _E
    rows_p = jnp.concatenate(
        [support_rows, jnp.zeros((_NS, pad), jnp.int32)], axis=1)
    cols_p = jnp.concatenate(
        [support_cols, jnp.zeros((_NS, pad), jnp.int32)], axis=1)
    vals_p = jnp.concatenate(
        [support_vals, jnp.zeros((_NS, pad), jnp.float32)], axis=1)
    lvl = jnp.arange(_NS, dtype=jnp.int32)[:, None]

    gidx = jnp.stack([cols_p * _NS + lvl,
                      _N * _NS + rows_p * _NS + lvl])
    lvl_local = jnp.where(lvl < _PASS_LVLS[0], lvl, lvl - _PASS_LVLS[0])
    sidx = jnp.stack([lvl_local * _N + rows_p,
                      lvl_local * _N + cols_p])
    gidx = gidx.reshape(2, _NS, _EP // 128, 128)
    sidx = sidx.reshape(2, _NS, _EP // 128, 128)
    vals_flat = vals_p.reshape(_NS * _EP)
    zeros = jnp.zeros((_ACC_ROWS // _NSUB, _DC), jnp.float32)

    out = _sc_aggregate(table, gidx, sidx, vals_flat, zeros)
    user = out[0].reshape(_NS, _N, _DC).transpose(1, 0, 2).reshape(_N, _DOUT)
    item = out[1].reshape(_NS, _N, _DC).transpose(1, 0, 2).reshape(_N, _DOUT)
    return (user, item)
